# Initial kernel scaffold; baseline (speedup 1.0000x reference)
import jax
import jax.numpy as jnp
from jax.experimental import pallas as pl


def kernel(inputs, W0, W1, W2, W3, W4, W5):
    raise NotImplementedError("write your pallas kernel here")


# speedup vs baseline: 1.5760x; 1.5760x over previous
import functools

import jax
import jax.numpy as jnp
from jax import lax
from jax.experimental import pallas as pl
from jax.experimental.pallas import tpu as pltpu
from jax.experimental.pallas import tpu_sc as plsc

DIM = 768
ROWS = 4096
NC, NS = 2, 16
NW = NC * NS
PER_W = ROWS // NW
CHUNK = 64
NCH = PER_W // CHUNK
NTAB = 6


def _build():
    mesh = plsc.VectorSubcoreMesh(core_axis_name="c", subcore_axis_name="s")
    out_type = [jax.ShapeDtypeStruct((ROWS, DIM), jnp.float32)] * NTAB
    scratch = [
        pltpu.VMEM((NCH, CHUNK), jnp.int32),
        pltpu.VMEM((CHUNK, DIM), jnp.float32),
        pltpu.VMEM((CHUNK, DIM), jnp.float32),
        pltpu.SemaphoreType.DMA,
        pltpu.SemaphoreType.DMA,
    ]

    @functools.partial(pl.kernel, mesh=mesh, out_type=out_type,
                       scratch_types=scratch)
    def gather6(idx_hbm, t0, t1, t2, t3, t4, t5,
                o0, o1, o2, o3, o4, o5,
                idx_v, buf_a, buf_b, sem_a, sem_b):
        tabs = [t0, t1, t2, t3, t4, t5]
        outs = [o0, o1, o2, o3, o4, o5]
        bufs = [buf_a, buf_b]
        sems = [sem_a, sem_b]
        wid = lax.axis_index("s") * NC + lax.axis_index("c")
        base = wid * PER_W
        pltpu.sync_copy(idx_hbm.at[pl.ds(wid * NCH, NCH)], idx_v)

        total = NTAB * NCH
        handles = {}

        def start(c):
            t, h = divmod(c, NCH)
            handles[c] = pltpu.async_copy(
                tabs[t].at[idx_v.at[h]], bufs[c % 2], sems[c % 2])

        start(0)
        start(1)
        for c in range(total):
            t, h = divmod(c, NCH)
            handles[c].wait()
            pltpu.sync_copy(bufs[c % 2],
                            outs[t].at[pl.ds(base + h * CHUNK, CHUNK)])
            if c + 2 < total:
                start(c + 2)

    return gather6


_GATHER6 = _build()


def kernel(inputs, W0, W1, W2, W3, W4, W5):
    b, s = inputs.shape
    idx = inputs.reshape(NW * NCH, CHUNK).astype(jnp.int32)
    outs = _GATHER6(idx, W0, W1, W2, W3, W4, W5)
    outs = [o.reshape(b, s, DIM) for o in outs]
    return tuple(outs) + tuple(outs[::-1])


# speedup vs baseline: 1.5793x; 1.0021x over previous
import functools

import jax
import jax.numpy as jnp
from jax import lax
from jax.experimental import pallas as pl
from jax.experimental.pallas import tpu as pltpu
from jax.experimental.pallas import tpu_sc as plsc

DIM = 768
ROWS = 4096
NC, NS = 2, 16
NW = NC * NS
PER_W = ROWS // NW
CHUNK = 32
NCH = PER_W // CHUNK
NTAB = 6
NBUF = 4


def _build():
    mesh = plsc.VectorSubcoreMesh(core_axis_name="c", subcore_axis_name="s")
    out_type = [jax.ShapeDtypeStruct((ROWS, DIM), jnp.float32)] * NTAB
    scratch = (
        [pltpu.VMEM((NCH, CHUNK), jnp.int32)]
        + [pltpu.VMEM((CHUNK, DIM), jnp.float32)] * NBUF
        + [pltpu.SemaphoreType.DMA] * (2 * NBUF)
    )

    @functools.partial(pl.kernel, mesh=mesh, out_type=out_type,
                       scratch_types=scratch)
    def gather6(idx_hbm, t0, t1, t2, t3, t4, t5,
                o0, o1, o2, o3, o4, o5,
                idx_v, *bufs_sems):
        tabs = [t0, t1, t2, t3, t4, t5]
        outs = [o0, o1, o2, o3, o4, o5]
        bufs = list(bufs_sems[:NBUF])
        gsems = list(bufs_sems[NBUF:2 * NBUF])
        ssems = list(bufs_sems[2 * NBUF:])
        wid = lax.axis_index("s") * NC + lax.axis_index("c")
        base = wid * PER_W
        pltpu.sync_copy(idx_hbm.at[pl.ds(wid * NCH, NCH)], idx_v)

        total = NTAB * NCH
        ghandles = {}
        shandles = {}

        def start_gather(c):
            t, h = divmod(c, NCH)
            b = c % NBUF
            ghandles[c] = pltpu.async_copy(
                tabs[t].at[idx_v.at[h]], bufs[b], gsems[b])

        def start_scatter(c):
            t, h = divmod(c, NCH)
            b = c % NBUF
            shandles[c] = pltpu.async_copy(
                bufs[b], outs[t].at[pl.ds(base + h * CHUNK, CHUNK)], ssems[b])

        for c in range(NBUF - 1):
            start_gather(c)
        for c in range(total):
            ghandles[c].wait()
            start_scatter(c)
            n = c + NBUF - 1
            if n < total:
                if n >= NBUF:
                    shandles[n - NBUF].wait()
                start_gather(n)
        for c in range(total - NBUF, total):
            shandles[c].wait()

    return gather6


_GATHER6 = _build()


def kernel(inputs, W0, W1, W2, W3, W4, W5):
    b, s = inputs.shape
    idx = inputs.reshape(NW * NCH, CHUNK).astype(jnp.int32)
    outs = _GATHER6(idx, W0, W1, W2, W3, W4, W5)
    outs = [o.reshape(b, s, DIM) for o in outs]
    return tuple(outs) + tuple(outs[::-1])


# speedup vs baseline: 2.0379x; 1.2904x over previous
import functools

import jax
import jax.numpy as jnp
from jax import lax
from jax.experimental import pallas as pl
from jax.experimental.pallas import tpu as pltpu
from jax.experimental.pallas import tpu_sc as plsc

DIM = 768
ROWS = 4096
NC, NS = 2, 16
NW = NC * NS
PER_W = ROWS // NW
CHUNK = 32
NCH = PER_W // CHUNK
NTAB = 6
NBUF = 4


def _build():
    mesh = plsc.VectorSubcoreMesh(core_axis_name="c", subcore_axis_name="s")
    out_type = [jax.ShapeDtypeStruct((ROWS, DIM), jnp.float32)] * (2 * NTAB)
    scratch = (
        [pltpu.VMEM((NCH, CHUNK), jnp.int32)]
        + [pltpu.VMEM((CHUNK, DIM), jnp.float32)] * NBUF
        + [pltpu.SemaphoreType.DMA] * (2 * NBUF)
    )

    @functools.partial(pl.kernel, mesh=mesh, out_type=out_type,
                       scratch_types=scratch)
    def gather12(idx_hbm, t0, t1, t2, t3, t4, t5, *rest):
        tabs = [t0, t1, t2, t3, t4, t5]
        outs = list(rest[:2 * NTAB])
        idx_v = rest[2 * NTAB]
        bufs = list(rest[2 * NTAB + 1:2 * NTAB + 1 + NBUF])
        gsems = list(rest[2 * NTAB + 1 + NBUF:2 * NTAB + 1 + 2 * NBUF])
        ssems = list(rest[2 * NTAB + 1 + 2 * NBUF:])
        wid = lax.axis_index("s") * NC + lax.axis_index("c")
        base = wid * PER_W
        pltpu.sync_copy(idx_hbm.at[pl.ds(wid * NCH, NCH)], idx_v)

        total = NTAB * NCH
        ghandles = {}
        shandles = {}

        def start_gather(c):
            t, h = divmod(c, NCH)
            b = c % NBUF
            ghandles[c] = pltpu.async_copy(
                tabs[t].at[idx_v.at[h]], bufs[b], gsems[b])

        def start_scatter(c):
            t, h = divmod(c, NCH)
            b = c % NBUF
            dst = pl.ds(base + h * CHUNK, CHUNK)
            shandles[c] = (
                pltpu.async_copy(bufs[b], outs[t].at[dst], ssems[b]),
                pltpu.async_copy(bufs[b], outs[11 - t].at[dst], ssems[b]),
            )

        def wait_scatter(c):
            shandles[c][0].wait()
            shandles[c][1].wait()

        for c in range(NBUF - 1):
            start_gather(c)
        for c in range(total):
            ghandles[c].wait()
            start_scatter(c)
            n = c + NBUF - 1
            if n < total:
                if n >= NBUF:
                    wait_scatter(n - NBUF)
                start_gather(n)
        for c in range(total - NBUF, total):
            wait_scatter(c)

    return gather12


_GATHER12 = _build()


def kernel(inputs, W0, W1, W2, W3, W4, W5):
    b, s = inputs.shape
    idx = inputs.reshape(NW * NCH, CHUNK).astype(jnp.int32)
    outs = _GATHER12(idx, W0, W1, W2, W3, W4, W5)
    return tuple(o.reshape(b, s, DIM) for o in outs)
